# baseline (speedup 1.0000x reference)
import functools
import math

import jax
import jax.numpy as jnp
from jax import lax
from jax.experimental import pallas as pl
from jax.experimental.pallas import tpu as pltpu
from jax.experimental.pallas import tpu_sc as plsc

_N = 512
_C = 96
_NBANDS = 32
_BAND = _N // _NBANDS
_PX = _BAND * _N
_W = 44
_WPX = _W * _N
_NC = 2
_NS = 16

_CUTMIN, _CUTMAX, _ALPHA = 2, 32, 1.0


def _build_field():
    n = _N
    beta_sample = 0.5
    cut = int(beta_sample * (_CUTMAX + 1 - _CUTMIN) + _CUTMIN)
    c_ = cut + 1e-06
    lg = math.log(c_)
    t1 = 1.0 / (math.pi * n ** 2 * lg)
    t2 = 4.0 / (math.pi ** 3 * c_ ** 2 * lg)
    t2 = max(t1, _ALPHA * t2)
    t = beta_sample * (t2 - t1) + t1

    def field(m, key):
        x = jnp.linspace(0.0, 1.0, n, dtype=jnp.float32)
        k = jnp.arange(1, m + 1, dtype=jnp.float32)
        i, j = jnp.meshgrid(k, k, indexing='ij')
        r = jnp.sqrt(i ** 2 + j ** 2)
        e = (r < m + 0.5).astype(jnp.float32) / r
        s = jnp.sin(jnp.pi * x[:, None] * k[None, :])
        c = jax.random.normal(key, (m, m), dtype=jnp.float32) * e
        return jnp.einsum('ij,xi,yj->yx', c, s, s)

    ku, kv = jax.random.split(jax.random.key(1))
    dx = (t ** 0.5) * field(cut, ku) * n
    dy = (t ** 0.5) * field(cut, kv) * n
    y, x = jnp.meshgrid(jnp.arange(n, dtype=jnp.float32),
                        jnp.arange(n, dtype=jnp.float32), indexing='ij')
    xn = jnp.clip(x - dx, 0.0, n - 1)
    yn = jnp.clip(y - dy, 0.0, n - 1)
    ybf = jnp.clip(jnp.floor(yn), 0, n - 2)
    xbf = jnp.clip(jnp.floor(xn), 0, n - 2)
    yv = yn - ybf
    xv = xn - xbf
    gbase = ybf.astype(jnp.int32) * n + xbf.astype(jnp.int32)
    return gbase, xv, yv


def _consts():
    gbase, xv, yv = _build_field()
    ybrow = gbase // _N
    ybmin = jnp.min(ybrow.reshape(_NBANDS, _PX), axis=1)
    lo = jnp.clip(ybmin - 2, 0, _N - _W).astype(jnp.int32)
    lo512 = lo * _N
    base = gbase.reshape(_NBANDS, _PX) - lo512[:, None]
    xq = jnp.round(xv * 65535.0).astype(jnp.uint32)
    yq = jnp.round(yv * 65535.0).astype(jnp.uint32)
    uv = jax.lax.bitcast_convert_type((yq << 16) | xq, jnp.int32)
    lo512_pad = jnp.concatenate([lo512, jnp.zeros(16, jnp.int32)])
    return base.reshape(-1), uv.reshape(-1), lo512_pad


def _make_remap():
    mesh = plsc.VectorSubcoreMesh(core_axis_name="c", subcore_axis_name="s")

    @functools.partial(
        pl.kernel,
        out_type=jax.ShapeDtypeStruct((_C * _N * _N,), jnp.float32),
        mesh=mesh,
        compiler_params=pltpu.CompilerParams(needs_layout_passes=False),
        scratch_types=[
            pltpu.VMEM((_PX,), jnp.int32),
            pltpu.VMEM((_PX,), jnp.int32),
            pltpu.VMEM((_NBANDS + 16,), jnp.int32),
            pltpu.VMEM((4 * _WPX,), jnp.float32),
            pltpu.VMEM((2 * _PX,), jnp.float32),
            pltpu.SemaphoreType.DMA,
            pltpu.SemaphoreType.DMA,
            pltpu.SemaphoreType.DMA,
            pltpu.SemaphoreType.DMA,
            pltpu.SemaphoreType.DMA,
            pltpu.SemaphoreType.DMA,
        ],
    )
    def _remap(img_hbm, base_hbm, uv_hbm, lo_hbm, out_hbm,
               base_v, uv_v, lo_v, win_v, out_v,
               sem_in00, sem_in01, sem_in10, sem_in11, sem_out0, sem_out1):
        cid = lax.axis_index("c")
        sid = lax.axis_index("s")
        wid = sid * _NC + cid
        mband = pl.multiple_of(wid * _PX, _PX)

        pltpu.sync_copy(base_hbm.at[pl.ds(mband, _PX)], base_v)
        pltpu.sync_copy(uv_hbm.at[pl.ds(mband, _PX)], uv_v)
        pltpu.sync_copy(lo_hbm, lo_v)
        lo512 = pl.multiple_of(lo_v[pl.ds(wid, 16)][0], _N)

        sems_in = ((sem_in00, sem_in01), (sem_in10, sem_in11))
        sems_out = (sem_out0, sem_out1)

        def in_copy(c, q, j):
            return pltpu.make_async_copy(
                img_hbm.at[pl.ds(pl.multiple_of(c * (_N * _N) + lo512, _N),
                                 _WPX)],
                win_v.at[pl.ds((2 * q + j) * _WPX, _WPX)],
                sems_in[q][j])

        def out_copy(c, j):
            return pltpu.make_async_copy(
                out_v.at[pl.ds(j * _PX, _PX)],
                out_hbm.at[pl.ds(pl.multiple_of(c * (_N * _N) + mband, _PX),
                                 _PX)],
                sems_out[j])

        in_copy(0, 0, 0).start()
        in_copy(1, 0, 1).start()

        def compute(q):
            win_a = win_v.at[pl.ds(2 * q * _WPX, _WPX)]
            win_b = win_v.at[pl.ds((2 * q + 1) * _WPX, _WPX)]

            @plsc.parallel_loop(0, _PX, step=16, unroll=4)
            def px(i):
                sl = pl.ds(i, 16)
                b = base_v[sl]
                i01 = b + 1
                i10 = b + _N
                i11 = b + (_N + 1)
                uvw = uv_v[sl]
                xv = (uvw & 0xFFFF).astype(jnp.float32) * (1.0 / 65535.0)
                yv = (lax.shift_right_logical(uvw, 16)
                      .astype(jnp.float32) * (1.0 / 65535.0))
                a00 = plsc.load_gather(win_a, [b])
                a01 = plsc.load_gather(win_a, [i01])
                a10 = plsc.load_gather(win_a, [i10])
                a11 = plsc.load_gather(win_a, [i11])
                atop = a00 + xv * (a01 - a00)
                abot = a10 + xv * (a11 - a10)
                out_v[pl.ds(i, 16)] = atop + yv * (abot - atop)
                c00 = plsc.load_gather(win_b, [b])
                c01 = plsc.load_gather(win_b, [i01])
                c10 = plsc.load_gather(win_b, [i10])
                c11 = plsc.load_gather(win_b, [i11])
                ctop = c00 + xv * (c01 - c00)
                cbot = c10 + xv * (c11 - c10)
                out_v[pl.ds(_PX + i, 16)] = ctop + yv * (cbot - ctop)

        def pair_duo(k2, _):
            for q in (0, 1):
                k = k2 * 2 + q
                c0 = 2 * k
                in_copy(c0, q, 0).wait()
                in_copy(c0 + 1, q, 1).wait()

                @pl.when(k < _C // 2 - 1)
                def _start_next():
                    in_copy(c0 + 2, 1 - q, 0).start()
                    in_copy(c0 + 3, 1 - q, 1).start()

                @pl.when(k >= 1)
                def _free_out():
                    out_copy(c0 - 2, 0).wait()
                    out_copy(c0 - 1, 1).wait()

                compute(q)
                out_copy(c0, 0).start()
                out_copy(c0 + 1, 1).start()
            return 0

        lax.fori_loop(0, _C // 4, pair_duo, 0)
        out_copy(_C - 2, 0).wait()
        out_copy(_C - 1, 1).wait()

    return _remap


_REMAP_CACHE = []


def kernel(img):
    base, uv, lo512 = _consts()
    if not _REMAP_CACHE:
        _REMAP_CACHE.append(_make_remap())
    img1 = img.reshape(-1)
    out = _REMAP_CACHE[0](img1, base, uv, lo512)
    return out.reshape(img.shape)


# speedup vs baseline: 1.1107x; 1.1107x over previous
import functools
import math

import jax
import jax.numpy as jnp
from jax import lax
from jax.experimental import pallas as pl
from jax.experimental.pallas import tpu as pltpu
from jax.experimental.pallas import tpu_sc as plsc

_N = 512
_C = 96
_NBANDS = 32
_BAND = _N // _NBANDS
_PX = _BAND * _N
_W = 44
_WPX = _W * _N
_NC = 2
_NS = 16

_CUTMIN, _CUTMAX, _ALPHA = 2, 32, 1.0


def _build_field():
    n = _N
    beta_sample = 0.5
    cut = int(beta_sample * (_CUTMAX + 1 - _CUTMIN) + _CUTMIN)
    c_ = cut + 1e-06
    lg = math.log(c_)
    t1 = 1.0 / (math.pi * n ** 2 * lg)
    t2 = 4.0 / (math.pi ** 3 * c_ ** 2 * lg)
    t2 = max(t1, _ALPHA * t2)
    t = beta_sample * (t2 - t1) + t1

    def field(m, key):
        x = jnp.linspace(0.0, 1.0, n, dtype=jnp.float32)
        k = jnp.arange(1, m + 1, dtype=jnp.float32)
        i, j = jnp.meshgrid(k, k, indexing='ij')
        r = jnp.sqrt(i ** 2 + j ** 2)
        e = (r < m + 0.5).astype(jnp.float32) / r
        s = jnp.sin(jnp.pi * x[:, None] * k[None, :])
        c = jax.random.normal(key, (m, m), dtype=jnp.float32) * e
        return jnp.einsum('ij,xi,yj->yx', c, s, s)

    ku, kv = jax.random.split(jax.random.key(1))
    dx = (t ** 0.5) * field(cut, ku) * n
    dy = (t ** 0.5) * field(cut, kv) * n
    y, x = jnp.meshgrid(jnp.arange(n, dtype=jnp.float32),
                        jnp.arange(n, dtype=jnp.float32), indexing='ij')
    xn = jnp.clip(x - dx, 0.0, n - 1)
    yn = jnp.clip(y - dy, 0.0, n - 1)
    ybf = jnp.clip(jnp.floor(yn), 0, n - 2)
    xbf = jnp.clip(jnp.floor(xn), 0, n - 2)
    yv = yn - ybf
    xv = xn - xbf
    gbase = ybf.astype(jnp.int32) * n + xbf.astype(jnp.int32)
    return gbase, xv, yv


def _consts():
    gbase, xv, yv = _build_field()
    ybrow = gbase // _N
    ybmin = jnp.min(ybrow.reshape(_NBANDS, _PX), axis=1)
    lo = jnp.clip(ybmin - 2, 0, _N - _W).astype(jnp.int32)
    lo512 = lo * _N
    base = gbase.reshape(_NBANDS, _PX) - lo512[:, None]
    xq = jnp.round(xv * 65535.0).astype(jnp.uint32)
    yq = jnp.round(yv * 65535.0).astype(jnp.uint32)
    uv = jax.lax.bitcast_convert_type((yq << 16) | xq, jnp.int32)
    lo512_pad = jnp.concatenate([lo512, jnp.zeros(16, jnp.int32)])
    return base.reshape(-1), uv.reshape(-1), lo512_pad


def _make_remap():
    mesh = plsc.VectorSubcoreMesh(core_axis_name="c", subcore_axis_name="s")

    @functools.partial(
        pl.kernel,
        out_type=jax.ShapeDtypeStruct((_C * _N * _N,), jnp.float32),
        mesh=mesh,
        compiler_params=pltpu.CompilerParams(needs_layout_passes=False),
        scratch_types=[
            pltpu.VMEM((_PX,), jnp.int32),
            pltpu.VMEM((_PX,), jnp.int32),
            pltpu.VMEM((_NBANDS + 16,), jnp.int32),
            pltpu.VMEM((4 * _WPX,), jnp.float32),
            pltpu.VMEM((2 * _PX,), jnp.float32),
            pltpu.SemaphoreType.DMA,
            pltpu.SemaphoreType.DMA,
            pltpu.SemaphoreType.DMA,
            pltpu.SemaphoreType.DMA,
            pltpu.SemaphoreType.DMA,
            pltpu.SemaphoreType.DMA,
        ],
    )
    def _remap(img_hbm, base_hbm, uv_hbm, lo_hbm, out_hbm,
               base_v, uv_v, lo_v, win_v, out_v,
               sem_in00, sem_in01, sem_in10, sem_in11, sem_out0, sem_out1):
        cid = lax.axis_index("c")
        sid = lax.axis_index("s")
        wid = sid * _NC + cid
        mband = pl.multiple_of(wid * _PX, _PX)

        pltpu.sync_copy(base_hbm.at[pl.ds(mband, _PX)], base_v)
        pltpu.sync_copy(uv_hbm.at[pl.ds(mband, _PX)], uv_v)
        pltpu.sync_copy(lo_hbm, lo_v)
        lo512 = pl.multiple_of(lo_v[pl.ds(wid, 16)][0], _N)

        sems_in = ((sem_in00, sem_in01), (sem_in10, sem_in11))
        sems_out = (sem_out0, sem_out1)

        def in_copy(c, q, j):
            return pltpu.make_async_copy(
                img_hbm.at[pl.ds(pl.multiple_of(c * (_N * _N) + lo512, _N),
                                 _WPX)],
                win_v.at[pl.ds((2 * q + j) * _WPX, _WPX)],
                sems_in[q][j])

        def out_copy(c, j):
            return pltpu.make_async_copy(
                out_v.at[pl.ds(j * _PX, _PX)],
                out_hbm.at[pl.ds(pl.multiple_of(c * (_N * _N) + mband, _PX),
                                 _PX)],
                sems_out[j])

        in_copy(0, 0, 0).start()
        in_copy(1, 0, 1).start()

        def compute(q):
            woff = 2 * q * _WPX

            @plsc.parallel_loop(0, _PX, step=16, unroll=4)
            def px(i):
                sl = pl.ds(i, 16)
                b = base_v[sl] + woff
                uvw = uv_v[sl]
                xv = (uvw & 0xFFFF).astype(jnp.float32) * (1.0 / 65535.0)
                yv = (lax.shift_right_logical(uvw, 16)
                      .astype(jnp.float32) * (1.0 / 65535.0))
                a00 = plsc.load_gather(win_v, [b])
                a01 = plsc.load_gather(win_v, [b + 1])
                a10 = plsc.load_gather(win_v, [b + _N])
                a11 = plsc.load_gather(win_v, [b + (_N + 1)])
                atop = a00 + xv * (a01 - a00)
                abot = a10 + xv * (a11 - a10)
                out_v[pl.ds(i, 16)] = atop + yv * (abot - atop)
                b2 = b + _WPX
                c00 = plsc.load_gather(win_v, [b2])
                c01 = plsc.load_gather(win_v, [b2 + 1])
                c10 = plsc.load_gather(win_v, [b2 + _N])
                c11 = plsc.load_gather(win_v, [b2 + (_N + 1)])
                ctop = c00 + xv * (c01 - c00)
                cbot = c10 + xv * (c11 - c10)
                out_v[pl.ds(_PX + i, 16)] = ctop + yv * (cbot - ctop)

        def pair_duo(k2, _):
            for q in (0, 1):
                k = k2 * 2 + q
                c0 = 2 * k
                in_copy(c0, q, 0).wait()
                in_copy(c0 + 1, q, 1).wait()

                @pl.when(k < _C // 2 - 1)
                def _start_next():
                    in_copy(c0 + 2, 1 - q, 0).start()
                    in_copy(c0 + 3, 1 - q, 1).start()

                @pl.when(k >= 1)
                def _free_out():
                    out_copy(c0 - 2, 0).wait()
                    out_copy(c0 - 1, 1).wait()

                compute(q)
                out_copy(c0, 0).start()
                out_copy(c0 + 1, 1).start()
            return 0

        lax.fori_loop(0, _C // 4, pair_duo, 0)
        out_copy(_C - 2, 0).wait()
        out_copy(_C - 1, 1).wait()

    return _remap


_REMAP_CACHE = []


def kernel(img):
    base, uv, lo512 = _consts()
    if not _REMAP_CACHE:
        _REMAP_CACHE.append(_make_remap())
    img1 = img.reshape(-1)
    out = _REMAP_CACHE[0](img1, base, uv, lo512)
    return out.reshape(img.shape)
